# baseline (speedup 1.0000x reference)
import functools

import jax
import jax.numpy as jnp
from jax import lax
from jax.experimental import pallas as pl
from jax.experimental.pallas import tpu as pltpu
from jax.experimental.pallas import tpu_sc as plsc

N = 10000
D = 128
E = 320000
RB = 1000
NT = 32
NSUB = 16
EPT = E // NT
CH = 80
NCH = EPT // CH
SG = 25
NGRP = NCH // SG
WSW = 16
NP = 10240
RPT = NP // NSUB
ZR = 128


def _mm_relu_body(h_ref, w_ref, b_ref, o_ref):
    acc = jnp.dot(h_ref[...], w_ref[...], preferred_element_type=jnp.float32)
    o_ref[...] = jax.nn.relu(acc + b_ref[...])


def _mm_relu(h, w, b):
    n = h.shape[0]
    return pl.pallas_call(
        _mm_relu_body,
        grid=(n // RB,),
        in_specs=[
            pl.BlockSpec((RB, D), lambda i: (i, 0)),
            pl.BlockSpec((D, D), lambda i: (0, 0)),
            pl.BlockSpec((1, D), lambda i: (0, 0)),
        ],
        out_specs=pl.BlockSpec((RB, D), lambda i: (i, 0)),
        out_shape=jax.ShapeDtypeStruct((n, D), jnp.float32),
    )(h, w, b.reshape(1, D))


def _combine_z(a0_ref, a1_ref, w0_ref, w1_ref, h_ref, wt_ref, wb_ref, b_ref):
    nagg = a0_ref[...] + a1_ref[...]
    ws = jnp.sum(w0_ref[...] + w1_ref[...], axis=1, keepdims=True)
    ws = jnp.clip(ws, 1.0, None)
    z = jnp.dot(nagg / ws, wt_ref[...], preferred_element_type=jnp.float32)
    z = z + jnp.dot(h_ref[...], wb_ref[...], preferred_element_type=jnp.float32)
    z = jax.nn.relu(z + b_ref[...])
    zn = jnp.sqrt(jnp.sum(z * z, axis=1, keepdims=True))
    zn = jnp.where(zn == 0.0, 1.0, zn)
    return z / zn


def _combine_body(a0_ref, a1_ref, w0_ref, w1_ref, h_ref, wt_ref, wb_ref,
                  b_ref, add_ref, o_ref):
    o_ref[...] = _combine_z(a0_ref, a1_ref, w0_ref, w1_ref, h_ref, wt_ref,
                            wb_ref, b_ref) + add_ref[...]


def _combine_mm_body(a0_ref, a1_ref, w0_ref, w1_ref, h_ref, wt_ref, wb_ref,
                     b_ref, qw_ref, qb_ref, o_ref, n_ref):
    hn = _combine_z(a0_ref, a1_ref, w0_ref, w1_ref, h_ref, wt_ref,
                    wb_ref, b_ref)
    o_ref[...] = hn
    acc = jnp.dot(hn, qw_ref[...], preferred_element_type=jnp.float32)
    n_ref[...] = jax.nn.relu(acc + qb_ref[...])


def _combine(a0, a1, w0, w1, h, Ww, Wb, add):
    nb = N // RB
    return pl.pallas_call(
        _combine_body,
        grid=(nb,),
        in_specs=[
            pl.BlockSpec((RB, D), lambda i: (i, 0)),
            pl.BlockSpec((RB, D), lambda i: (i, 0)),
            pl.BlockSpec((RB, WSW), lambda i: (i, 0)),
            pl.BlockSpec((RB, WSW), lambda i: (i, 0)),
            pl.BlockSpec((RB, D), lambda i: (i, 0)),
            pl.BlockSpec((D, D), lambda i: (0, 0)),
            pl.BlockSpec((D, D), lambda i: (0, 0)),
            pl.BlockSpec((1, D), lambda i: (0, 0)),
            pl.BlockSpec((RB, D), lambda i: (i, 0)),
        ],
        out_specs=pl.BlockSpec((RB, D), lambda i: (i, 0)),
        out_shape=jax.ShapeDtypeStruct((N, D), jnp.float32),
    )(a0, a1, w0, w1, h, Ww[:D], Ww[D:], Wb.reshape(1, D), add)


def _combine_mm(a0, a1, w0, w1, h, Ww, Wb, Qw, Qb):
    nb = N // RB
    return pl.pallas_call(
        _combine_mm_body,
        grid=(nb,),
        in_specs=[
            pl.BlockSpec((RB, D), lambda i: (i, 0)),
            pl.BlockSpec((RB, D), lambda i: (i, 0)),
            pl.BlockSpec((RB, WSW), lambda i: (i, 0)),
            pl.BlockSpec((RB, WSW), lambda i: (i, 0)),
            pl.BlockSpec((RB, D), lambda i: (i, 0)),
            pl.BlockSpec((D, D), lambda i: (0, 0)),
            pl.BlockSpec((D, D), lambda i: (0, 0)),
            pl.BlockSpec((1, D), lambda i: (0, 0)),
            pl.BlockSpec((D, D), lambda i: (0, 0)),
            pl.BlockSpec((1, D), lambda i: (0, 0)),
        ],
        out_specs=[
            pl.BlockSpec((RB, D), lambda i: (i, 0)),
            pl.BlockSpec((RB, D), lambda i: (i, 0)),
        ],
        out_shape=[
            jax.ShapeDtypeStruct((N, D), jnp.float32),
            jax.ShapeDtypeStruct((N, D), jnp.float32),
        ],
    )(a0, a1, w0, w1, h, Ww[:D], Ww[D:], Wb.reshape(1, D), Qw,
      Qb.reshape(1, D))


def _perm_lanes(v, idx):
    return lax.gather(
        v, idx,
        dimension_numbers=lax.GatherDimensionNumbers(
            offset_dims=(), collapsed_slice_dims=(0,), start_index_map=(0,)),
        slice_sizes=(1,),
        mode=lax.GatherScatterMode.PROMISE_IN_BOUNDS)


def _splat_lane(v, e):
    return _perm_lanes(v, jnp.full((16, 1), e, jnp.int32))


def _hsum_lanes(v, lane_iota):
    for m in (8, 4, 2, 1):
        v = v + _perm_lanes(v, (lane_iota ^ m)[:, None])
    return v

def _edge_body(nsrc_hbm, src_hbm, dst_hbm, w_hbm, acc_out, ws_out,
               src_v, dst_v, w_v, rows0, rows1, rows2, wblk_v,
               acc_sh, ws_sh, gsem, ssem, wsem):
    c = lax.axis_index("c")
    s = lax.axis_index("s")
    t = c * NSUB + s
    zero16 = jnp.zeros((16,), jnp.float32)
    lane_iota = lax.iota(jnp.int32, 16)
    rowsb = [rows0, rows1, rows2]

    def _zrow(i, _):
        for dd in range(D // 16):
            rows0[i, pl.ds(dd * 16, 16)] = zero16
        wblk_v[i, :] = zero16
        return 0
    lax.fori_loop(0, CH, _zrow, 0)

    for k in range(RPT // CH):
        pltpu.sync_copy(rows0, acc_sh.at[pl.ds(s * RPT + k * CH, CH)])
        pltpu.sync_copy(wblk_v, ws_sh.at[pl.ds(s * RPT + k * CH, CH)])

    plsc.subcore_barrier()

    def _scale(rows, cc):
        def _sixteen(g, _):
            wv = w_v[cc, pl.ds(g * 16, 16)]
            for e in range(16):
                r = g * 16 + e
                wsp = _splat_lane(wv, e)
                wblk_v[r, :] = jnp.where(lane_iota == 0, wsp, 0.0)
                for dd in range(D // 16):
                    sl = pl.ds(dd * 16, 16)
                    rows[r, sl] = rows[r, sl] * wsp
            return 0
        lax.fori_loop(0, CH // 16, _sixteen, 0)

    def _substep(cc, b, bn, issue_next, wait_ssem, wait_wsem=True):
        rows = rowsb[b]
        pltpu.make_async_copy(nsrc_hbm.at[src_v.at[cc]], rows, gsem.at[b]).wait()
        if wait_wsem:
            pltpu.make_async_copy(wblk_v, ws_sh.at[dst_v.at[0]], wsem).wait()
        _scale(rows, cc)
        pltpu.async_copy(wblk_v, ws_sh.at[dst_v.at[cc]], wsem, add=True)
        if issue_next:
            if wait_ssem:
                pltpu.make_async_copy(
                    rowsb[bn], acc_sh.at[dst_v.at[0]], ssem.at[bn]).wait()
            pltpu.async_copy(nsrc_hbm.at[src_v.at[cc + 2]], rowsb[bn],
                             gsem.at[bn])
        pltpu.async_copy(rows, acc_sh.at[dst_v.at[cc]], ssem.at[b], add=True)

    def _group(g, _):
        base = t * NGRP + g
        pltpu.sync_copy(src_hbm.at[base], src_v)
        pltpu.sync_copy(dst_hbm.at[base], dst_v)
        pltpu.sync_copy(w_hbm.at[base], w_v)
        pltpu.async_copy(nsrc_hbm.at[src_v.at[0]], rowsb[0], gsem.at[0])
        pltpu.async_copy(nsrc_hbm.at[src_v.at[1]], rowsb[1], gsem.at[1])
        _substep(0, 0, 2, True, False, wait_wsem=False)

        def _triple(jj, _):
            cb = 3 * jj
            _substep(cb + 1, 1, 0, True, True)
            _substep(cb + 2, 2, 1, True, True)
            _substep(cb + 3, 0, 2, True, True)
            return 0
        lax.fori_loop(0, 7, _triple, 0)

        _substep(SG - 3, 1, 0, True, True)
        _substep(SG - 2, 2, 1, False, False)
        _substep(SG - 1, 0, 2, False, False)
        for b in range(3):
            pltpu.make_async_copy(
                rowsb[b], acc_sh.at[dst_v.at[0]], ssem.at[b]).wait()
        pltpu.make_async_copy(wblk_v, ws_sh.at[dst_v.at[0]], wsem).wait()
        return 0

    lax.fori_loop(0, NGRP, _group, 0)

    plsc.subcore_barrier()

    for k in range(RPT // CH):
        pltpu.sync_copy(acc_sh.at[pl.ds(s * RPT + k * CH, CH)], rows0)
        pltpu.sync_copy(rows0, acc_out.at[pl.ds(c * NP + s * RPT + k * CH, CH)])
        pltpu.sync_copy(ws_sh.at[pl.ds(s * RPT + k * CH, CH)], wblk_v)
        pltpu.sync_copy(wblk_v, ws_out.at[pl.ds(c * NP + s * RPT + k * CH, CH)])


@jax.jit
def _edge_agg(n_src, src_r, dst_r, w_r):
    mesh = plsc.VectorSubcoreMesh(core_axis_name="c", subcore_axis_name="s")
    f = pl.kernel(
        _edge_body,
        out_type=(
            jax.ShapeDtypeStruct((2 * NP, D), jnp.float32),
            jax.ShapeDtypeStruct((2 * NP, WSW), jnp.float32),
        ),
        mesh=mesh,
        scratch_types=[
            pltpu.VMEM((SG, CH), jnp.int32),
            pltpu.VMEM((SG, CH), jnp.int32),
            pltpu.VMEM((SG, CH), jnp.float32),
            pltpu.VMEM((CH, D), jnp.float32),
            pltpu.VMEM((CH, D), jnp.float32),
            pltpu.VMEM((CH, D), jnp.float32),
            pltpu.VMEM((CH, WSW), jnp.float32),
            pltpu.VMEM_SHARED((NP, D), jnp.float32),
            pltpu.VMEM_SHARED((NP, WSW), jnp.float32),
            pltpu.SemaphoreType.DMA((3,)),
            pltpu.SemaphoreType.DMA((3,)),
            pltpu.SemaphoreType.DMA,
        ],
        compiler_params=pltpu.CompilerParams(use_tc_tiling_on_sc=False),
    )
    return f(n_src, src_r, dst_r, w_r)


NCK = 5
CS = NP // NT // NCK


def _gather_body(emb_hbm, bias_hbm, nids_hbm, h0_out, bn_out,
                 nids_v, r0, r1, r2, r3, r4, b_v, gsem, bsem):
    c = lax.axis_index("c")
    s = lax.axis_index("s")
    t = c * NSUB + s
    rbufs = [r0, r1, r2, r3, r4]
    pltpu.sync_copy(nids_hbm.at[t], nids_v)
    for k in range(NCK):
        pltpu.async_copy(emb_hbm.at[nids_v.at[k]], rbufs[k], gsem)
        pltpu.async_copy(bias_hbm.at[nids_v.at[k]], b_v.at[k], bsem)
    for k in range(NCK):
        pltpu.make_async_copy(emb_hbm.at[nids_v.at[k]], rbufs[k], gsem).wait()
        pltpu.sync_copy(rbufs[k], h0_out.at[pl.ds(t * NCK * CS + k * CS, CS)])
    for k in range(NCK):
        pltpu.make_async_copy(bias_hbm.at[nids_v.at[k]], b_v.at[k], bsem).wait()
    pltpu.sync_copy(b_v, bn_out.at[pl.ds(t * NCK, NCK)])


@jax.jit
def _gather_h0(emb_table, bias1d, nids_p):
    mesh = plsc.VectorSubcoreMesh(core_axis_name="c", subcore_axis_name="s")
    f = pl.kernel(
        _gather_body,
        out_type=(
            jax.ShapeDtypeStruct((NP, D), jnp.float32),
            jax.ShapeDtypeStruct((NP // CS, CS), jnp.float32),
        ),
        mesh=mesh,
        scratch_types=[
            pltpu.VMEM((NCK, CS), jnp.int32),
            pltpu.VMEM((CS, D), jnp.float32),
            pltpu.VMEM((CS, D), jnp.float32),
            pltpu.VMEM((CS, D), jnp.float32),
            pltpu.VMEM((CS, D), jnp.float32),
            pltpu.VMEM((CS, D), jnp.float32),
            pltpu.VMEM((NCK, CS), jnp.float32),
            pltpu.SemaphoreType.DMA,
            pltpu.SemaphoreType.DMA,
        ],
        compiler_params=pltpu.CompilerParams(use_tc_tiling_on_sc=False),
    )
    return f(emb_table, bias1d, nids_p)


def _score_body(hitem_hbm, bn_hbm, e0_hbm, e1_hbm, u_out, v_out, b0_out, b1_out,
                e0_v, e1_v, u0, v0, u1, v1, b0_v, b1_v, gsem, bsem):
    c = lax.axis_index("c")
    s = lax.axis_index("s")
    t = c * NSUB + s
    ubufs = [u0, u1]
    vbufs = [v0, v1]
    pltpu.sync_copy(e0_hbm.at[t], e0_v)
    pltpu.sync_copy(e1_hbm.at[t], e1_v)
    for k in range(NCK):
        pltpu.async_copy(bn_hbm.at[e0_v.at[k]], b0_v.at[k], bsem)
        pltpu.async_copy(bn_hbm.at[e1_v.at[k]], b1_v.at[k], bsem)
    pltpu.async_copy(hitem_hbm.at[e0_v.at[0]], ubufs[0], gsem)
    pltpu.async_copy(hitem_hbm.at[e1_v.at[0]], vbufs[0], gsem)
    for k in range(NCK):
        pltpu.make_async_copy(bn_hbm.at[e0_v.at[k]], b0_v.at[k], bsem).wait()
        pltpu.make_async_copy(bn_hbm.at[e1_v.at[k]], b1_v.at[k], bsem).wait()
    pltpu.sync_copy(b0_v, b0_out.at[pl.ds(t * NCK, NCK)])
    pltpu.sync_copy(b1_v, b1_out.at[pl.ds(t * NCK, NCK)])
    for k in range(NCK):
        bb = k % 2
        u, v = ubufs[bb], vbufs[bb]
        pltpu.make_async_copy(hitem_hbm.at[e0_v.at[k]], u, gsem).wait()
        pltpu.make_async_copy(hitem_hbm.at[e1_v.at[k]], v, gsem).wait()
        if k + 1 < NCK:
            nb = (k + 1) % 2
            pltpu.async_copy(hitem_hbm.at[e0_v.at[k + 1]], ubufs[nb], gsem)
            pltpu.async_copy(hitem_hbm.at[e1_v.at[k + 1]], vbufs[nb], gsem)
        pltpu.sync_copy(u, u_out.at[pl.ds(t * 640 + k * 128, 128)])
        pltpu.sync_copy(v, v_out.at[pl.ds(t * 640 + k * 128, 128)])


@jax.jit
def _score_edges(h_item, bn, e0r, e1r):
    mesh = plsc.VectorSubcoreMesh(core_axis_name="c", subcore_axis_name="s")
    f = pl.kernel(
        _score_body,
        out_type=(
            jax.ShapeDtypeStruct((2 * NP, D), jnp.float32),
            jax.ShapeDtypeStruct((2 * NP, D), jnp.float32),
            jax.ShapeDtypeStruct((NT * NCK, 128), jnp.float32),
            jax.ShapeDtypeStruct((NT * NCK, 128), jnp.float32),
        ),
        mesh=mesh,
        scratch_types=[
            pltpu.VMEM((NCK, 128), jnp.int32),
            pltpu.VMEM((NCK, 128), jnp.int32),
            pltpu.VMEM((128, D), jnp.float32),
            pltpu.VMEM((128, D), jnp.float32),
            pltpu.VMEM((128, D), jnp.float32),
            pltpu.VMEM((128, D), jnp.float32),
            pltpu.VMEM((NCK, 128), jnp.float32),
            pltpu.VMEM((NCK, 128), jnp.float32),
            pltpu.SemaphoreType.DMA,
            pltpu.SemaphoreType.DMA,
        ],
        compiler_params=pltpu.CompilerParams(use_tc_tiling_on_sc=False),
    )
    return f(h_item, bn, e0r, e1r)


def _dot_body(u_ref, v_ref, b0_ref, b1_ref, ones_ref, o_ref):
    m = u_ref[...] * v_ref[...]
    s = jnp.dot(m, ones_ref[...], preferred_element_type=jnp.float32)
    o_ref[...] = s + b0_ref[...] + b1_ref[...]


def _dot_scores(U, V, B0, B1):
    PE = U.shape[0]
    RB2 = 1024
    ones = jnp.ones((D, 1), jnp.float32)
    return pl.pallas_call(
        _dot_body,
        grid=(PE // RB2,),
        in_specs=[
            pl.BlockSpec((RB2, D), lambda i: (i, 0)),
            pl.BlockSpec((RB2, D), lambda i: (i, 0)),
            pl.BlockSpec((RB2, 1), lambda i: (i, 0)),
            pl.BlockSpec((RB2, 1), lambda i: (i, 0)),
            pl.BlockSpec((D, 1), lambda i: (0, 0)),
        ],
        out_specs=pl.BlockSpec((RB2, 1), lambda i: (i, 0)),
        out_shape=jax.ShapeDtypeStruct((PE, 1), jnp.float32),
    )(U, V, B0.reshape(PE, 1), B1.reshape(PE, 1), ones)


def kernel(nids, edge_index1, weights1, edge_index2, weights2, pos_edges, neg_edges,
           emb_table, bias, Q1w, Q1b, W1w, W1b, Q2w, Q2b, W2w, W2b):
    nids_p = jnp.concatenate(
        [nids.astype(jnp.int32), jnp.zeros((NP - N,), jnp.int32)]
    ).reshape(NT, NCK, CS)
    h0p, bn2 = _gather_h0(emb_table, bias.reshape(N), nids_p)

    esh = (NT * NGRP, SG, CH)
    s1, d1 = edge_index1[0].reshape(esh), edge_index1[1].reshape(esh)
    s2, d2 = edge_index2[0].reshape(esh), edge_index2[1].reshape(esh)
    w1r = weights1.reshape(esh)
    w2r = weights2.reshape(esh)

    nsrc1 = _mm_relu(h0p, Q1w, Q1b)
    acc2, ws2 = _edge_agg(nsrc1, s1, d1, w1r)
    a = acc2.reshape(2, NP, D)
    w = ws2.reshape(2, NP, WSW)
    h1, nsrc2 = _combine_mm(a[0], a[1], w[0], w[1], h0p, W1w, W1b, Q2w, Q2b)
    acc2b, ws2b = _edge_agg(nsrc2, s2, d2, w2r)
    ab = acc2b.reshape(2, NP, D)
    wb = ws2b.reshape(2, NP, WSW)
    h_item = _combine(ab[0], ab[1], wb[0], wb[1], h1, W2w, W2b, h0p)

    P = pos_edges.shape[1]
    ep = jnp.concatenate([pos_edges, neg_edges], axis=1).astype(jnp.int32)
    ep = jnp.concatenate(
        [ep, jnp.zeros((2, 2 * NP - 2 * P), jnp.int32)], axis=1)
    e0r = ep[0].reshape(NT, NCK, 128)
    e1r = ep[1].reshape(NT, NCK, 128)
    U, V, B0, B1 = _score_edges(h_item, bn2.reshape(NP), e0r, e1r)
    scores = _dot_scores(U, V, B0, B1)
    return scores[:2 * P]


# speedup vs baseline: 1.1164x; 1.1164x over previous
import functools

import jax
import jax.numpy as jnp
from jax import lax
from jax.experimental import pallas as pl
from jax.experimental.pallas import tpu as pltpu
from jax.experimental.pallas import tpu_sc as plsc

N = 10000
D = 128
E = 320000
RB = 1000
NT = 32
NSUB = 16
EPT = E // NT
CH = 80
NCH = EPT // CH
SG = 25
NGRP = NCH // SG
WSW = 16
NP = 10240
RPT = NP // NSUB
ZR = 128


def _mm_relu_body(h_ref, w_ref, b_ref, o_ref):
    acc = jnp.dot(h_ref[...], w_ref[...], preferred_element_type=jnp.float32)
    o_ref[...] = jax.nn.relu(acc + b_ref[...])


def _mm_relu(h, w, b):
    n = h.shape[0]
    return pl.pallas_call(
        _mm_relu_body,
        grid=(n // RB,),
        in_specs=[
            pl.BlockSpec((RB, D), lambda i: (i, 0)),
            pl.BlockSpec((D, D), lambda i: (0, 0)),
            pl.BlockSpec((1, D), lambda i: (0, 0)),
        ],
        out_specs=pl.BlockSpec((RB, D), lambda i: (i, 0)),
        out_shape=jax.ShapeDtypeStruct((n, D), jnp.float32),
    )(h, w, b.reshape(1, D))


def _combine_z(a0_ref, a1_ref, w0_ref, w1_ref, h_ref, wt_ref, wb_ref, b_ref):
    nagg = a0_ref[...] + a1_ref[...]
    ws = jnp.sum(w0_ref[...] + w1_ref[...], axis=1, keepdims=True)
    ws = jnp.clip(ws, 1.0, None)
    z = jnp.dot(nagg / ws, wt_ref[...], preferred_element_type=jnp.float32)
    z = z + jnp.dot(h_ref[...], wb_ref[...], preferred_element_type=jnp.float32)
    z = jax.nn.relu(z + b_ref[...])
    zn = jnp.sqrt(jnp.sum(z * z, axis=1, keepdims=True))
    zn = jnp.where(zn == 0.0, 1.0, zn)
    return z / zn


def _combine_body(a0_ref, a1_ref, w0_ref, w1_ref, h_ref, wt_ref, wb_ref,
                  b_ref, add_ref, o_ref):
    o_ref[...] = _combine_z(a0_ref, a1_ref, w0_ref, w1_ref, h_ref, wt_ref,
                            wb_ref, b_ref) + add_ref[...]


def _combine_mm_body(a0_ref, a1_ref, w0_ref, w1_ref, h_ref, wt_ref, wb_ref,
                     b_ref, qw_ref, qb_ref, o_ref, n_ref):
    hn = _combine_z(a0_ref, a1_ref, w0_ref, w1_ref, h_ref, wt_ref,
                    wb_ref, b_ref)
    o_ref[...] = hn
    acc = jnp.dot(hn, qw_ref[...], preferred_element_type=jnp.float32)
    n_ref[...] = jax.nn.relu(acc + qb_ref[...])


def _combine(a0, a1, w0, w1, h, Ww, Wb, add):
    nb = N // RB
    return pl.pallas_call(
        _combine_body,
        grid=(nb,),
        in_specs=[
            pl.BlockSpec((RB, D), lambda i: (i, 0)),
            pl.BlockSpec((RB, D), lambda i: (i, 0)),
            pl.BlockSpec((RB, WSW), lambda i: (i, 0)),
            pl.BlockSpec((RB, WSW), lambda i: (i, 0)),
            pl.BlockSpec((RB, D), lambda i: (i, 0)),
            pl.BlockSpec((D, D), lambda i: (0, 0)),
            pl.BlockSpec((D, D), lambda i: (0, 0)),
            pl.BlockSpec((1, D), lambda i: (0, 0)),
            pl.BlockSpec((RB, D), lambda i: (i, 0)),
        ],
        out_specs=pl.BlockSpec((RB, D), lambda i: (i, 0)),
        out_shape=jax.ShapeDtypeStruct((N, D), jnp.float32),
    )(a0, a1, w0, w1, h, Ww[:D], Ww[D:], Wb.reshape(1, D), add)


def _combine_mm(a0, a1, w0, w1, h, Ww, Wb, Qw, Qb):
    nb = N // RB
    return pl.pallas_call(
        _combine_mm_body,
        grid=(nb,),
        in_specs=[
            pl.BlockSpec((RB, D), lambda i: (i, 0)),
            pl.BlockSpec((RB, D), lambda i: (i, 0)),
            pl.BlockSpec((RB, WSW), lambda i: (i, 0)),
            pl.BlockSpec((RB, WSW), lambda i: (i, 0)),
            pl.BlockSpec((RB, D), lambda i: (i, 0)),
            pl.BlockSpec((D, D), lambda i: (0, 0)),
            pl.BlockSpec((D, D), lambda i: (0, 0)),
            pl.BlockSpec((1, D), lambda i: (0, 0)),
            pl.BlockSpec((D, D), lambda i: (0, 0)),
            pl.BlockSpec((1, D), lambda i: (0, 0)),
        ],
        out_specs=[
            pl.BlockSpec((RB, D), lambda i: (i, 0)),
            pl.BlockSpec((RB, D), lambda i: (i, 0)),
        ],
        out_shape=[
            jax.ShapeDtypeStruct((N, D), jnp.float32),
            jax.ShapeDtypeStruct((N, D), jnp.float32),
        ],
    )(a0, a1, w0, w1, h, Ww[:D], Ww[D:], Wb.reshape(1, D), Qw,
      Qb.reshape(1, D))


def _perm_lanes(v, idx):
    return lax.gather(
        v, idx,
        dimension_numbers=lax.GatherDimensionNumbers(
            offset_dims=(), collapsed_slice_dims=(0,), start_index_map=(0,)),
        slice_sizes=(1,),
        mode=lax.GatherScatterMode.PROMISE_IN_BOUNDS)


def _splat_lane(v, e):
    return _perm_lanes(v, jnp.full((16, 1), e, jnp.int32))


def _hsum_lanes(v, lane_iota):
    for m in (8, 4, 2, 1):
        v = v + _perm_lanes(v, (lane_iota ^ m)[:, None])
    return v

def _edge_body(nsrc_hbm, src_hbm, dst_hbm, w_hbm, acc_out, ws_out,
               src_v, dst_v, w_v, rows0, rows1, rows2, wblk_v,
               acc_sh, ws_sh, gsem, ssem, wsem):
    c = lax.axis_index("c")
    s = lax.axis_index("s")
    t = c * NSUB + s
    zero16 = jnp.zeros((16,), jnp.float32)
    lane_iota = lax.iota(jnp.int32, 16)
    rowsb = [rows0, rows1, rows2]

    def _zrow(i, _):
        for dd in range(D // 16):
            rows0[i, pl.ds(dd * 16, 16)] = zero16
        wblk_v[i, :] = zero16
        return 0
    lax.fori_loop(0, CH, _zrow, 0)

    for k in range(RPT // CH):
        pltpu.sync_copy(rows0, acc_sh.at[pl.ds(s * RPT + k * CH, CH)])
        pltpu.sync_copy(wblk_v, ws_sh.at[pl.ds(s * RPT + k * CH, CH)])

    plsc.subcore_barrier()

    def _scale(rows, cc):
        def _sixteen(g, _):
            wv = w_v[cc, pl.ds(g * 16, 16)]
            for e in range(16):
                r = g * 16 + e
                wsp = _splat_lane(wv, e)
                wblk_v[r, :] = jnp.where(lane_iota == 0, wsp, 0.0)
                for dd in range(D // 16):
                    sl = pl.ds(dd * 16, 16)
                    rows[r, sl] = rows[r, sl] * wsp
            return 0
        lax.fori_loop(0, CH // 16, _sixteen, 0)

    def _substep(cc, b, bn, issue_next, wait_ssem, wait_wsem=True):
        rows = rowsb[b]
        pltpu.make_async_copy(nsrc_hbm.at[src_v.at[cc]], rows, gsem.at[b]).wait()
        if wait_wsem:
            pltpu.make_async_copy(wblk_v, ws_sh.at[dst_v.at[0]], wsem).wait()
        _scale(rows, cc)
        pltpu.async_copy(wblk_v, ws_sh.at[dst_v.at[cc]], wsem, add=True)
        if issue_next:
            if wait_ssem:
                pltpu.make_async_copy(
                    rowsb[bn], acc_sh.at[dst_v.at[0]], ssem.at[bn]).wait()
            pltpu.async_copy(nsrc_hbm.at[src_v.at[cc + 2]], rowsb[bn],
                             gsem.at[bn])
        pltpu.async_copy(rows, acc_sh.at[dst_v.at[cc]], ssem.at[b], add=True)

    def _group(g, _):
        base = t * NGRP + g
        pltpu.sync_copy(src_hbm.at[base], src_v)
        pltpu.sync_copy(dst_hbm.at[base], dst_v)
        pltpu.sync_copy(w_hbm.at[base], w_v)
        pltpu.async_copy(nsrc_hbm.at[src_v.at[0]], rowsb[0], gsem.at[0])
        pltpu.async_copy(nsrc_hbm.at[src_v.at[1]], rowsb[1], gsem.at[1])
        _substep(0, 0, 2, True, False, wait_wsem=False)

        def _triple(jj, _):
            cb = 3 * jj
            _substep(cb + 1, 1, 0, True, True)
            _substep(cb + 2, 2, 1, True, True)
            _substep(cb + 3, 0, 2, True, True)
            return 0
        lax.fori_loop(0, 7, _triple, 0)

        _substep(SG - 3, 1, 0, True, True)
        _substep(SG - 2, 2, 1, False, False)
        _substep(SG - 1, 0, 2, False, False)
        for b in range(3):
            pltpu.make_async_copy(
                rowsb[b], acc_sh.at[dst_v.at[0]], ssem.at[b]).wait()
        pltpu.make_async_copy(wblk_v, ws_sh.at[dst_v.at[0]], wsem).wait()
        return 0

    lax.fori_loop(0, NGRP, _group, 0)

    plsc.subcore_barrier()

    for k in range(RPT // CH):
        pltpu.sync_copy(acc_sh.at[pl.ds(s * RPT + k * CH, CH)], rows0)
        pltpu.sync_copy(rows0, acc_out.at[pl.ds(c * NP + s * RPT + k * CH, CH)])
        pltpu.sync_copy(ws_sh.at[pl.ds(s * RPT + k * CH, CH)], wblk_v)
        pltpu.sync_copy(wblk_v, ws_out.at[pl.ds(c * NP + s * RPT + k * CH, CH)])


@jax.jit
def _edge_agg(n_src, src_r, dst_r, w_r):
    mesh = plsc.VectorSubcoreMesh(core_axis_name="c", subcore_axis_name="s")
    f = pl.kernel(
        _edge_body,
        out_type=(
            jax.ShapeDtypeStruct((2 * NP, D), jnp.float32),
            jax.ShapeDtypeStruct((2 * NP, WSW), jnp.float32),
        ),
        mesh=mesh,
        scratch_types=[
            pltpu.VMEM((SG, CH), jnp.int32),
            pltpu.VMEM((SG, CH), jnp.int32),
            pltpu.VMEM((SG, CH), jnp.float32),
            pltpu.VMEM((CH, D), jnp.float32),
            pltpu.VMEM((CH, D), jnp.float32),
            pltpu.VMEM((CH, D), jnp.float32),
            pltpu.VMEM((CH, WSW), jnp.float32),
            pltpu.VMEM_SHARED((NP, D), jnp.float32),
            pltpu.VMEM_SHARED((NP, WSW), jnp.float32),
            pltpu.SemaphoreType.DMA((3,)),
            pltpu.SemaphoreType.DMA((3,)),
            pltpu.SemaphoreType.DMA,
        ],
        compiler_params=pltpu.CompilerParams(use_tc_tiling_on_sc=False),
    )
    return f(n_src, src_r, dst_r, w_r)


NCK = 5
CS = NP // NT // NCK


def _gather_body(emb_hbm, bias_hbm, nids_hbm, h0_out, bn_out,
                 nids_v, r0, r1, r2, r3, r4, b_v, gsem, bsem):
    c = lax.axis_index("c")
    s = lax.axis_index("s")
    t = c * NSUB + s
    rbufs = [r0, r1, r2, r3, r4]
    pltpu.sync_copy(nids_hbm.at[t], nids_v)
    for k in range(NCK):
        pltpu.async_copy(emb_hbm.at[nids_v.at[k]], rbufs[k], gsem)
        pltpu.async_copy(bias_hbm.at[nids_v.at[k]], b_v.at[k], bsem)
    for k in range(NCK):
        pltpu.make_async_copy(emb_hbm.at[nids_v.at[k]], rbufs[k], gsem).wait()
        pltpu.sync_copy(rbufs[k], h0_out.at[pl.ds(t * NCK * CS + k * CS, CS)])
    for k in range(NCK):
        pltpu.make_async_copy(bias_hbm.at[nids_v.at[k]], b_v.at[k], bsem).wait()
    pltpu.sync_copy(b_v, bn_out.at[pl.ds(t * NCK, NCK)])


@jax.jit
def _gather_h0(emb_table, bias1d, nids_p):
    mesh = plsc.VectorSubcoreMesh(core_axis_name="c", subcore_axis_name="s")
    f = pl.kernel(
        _gather_body,
        out_type=(
            jax.ShapeDtypeStruct((NP, D), jnp.float32),
            jax.ShapeDtypeStruct((NP // CS, CS), jnp.float32),
        ),
        mesh=mesh,
        scratch_types=[
            pltpu.VMEM((NCK, CS), jnp.int32),
            pltpu.VMEM((CS, D), jnp.float32),
            pltpu.VMEM((CS, D), jnp.float32),
            pltpu.VMEM((CS, D), jnp.float32),
            pltpu.VMEM((CS, D), jnp.float32),
            pltpu.VMEM((CS, D), jnp.float32),
            pltpu.VMEM((NCK, CS), jnp.float32),
            pltpu.SemaphoreType.DMA,
            pltpu.SemaphoreType.DMA,
        ],
        compiler_params=pltpu.CompilerParams(use_tc_tiling_on_sc=False),
    )
    return f(emb_table, bias1d, nids_p)


def _score_body(hitem_hbm, bn_hbm, e0_hbm, e1_hbm, out_hbm,
                e0_v, e1_v, u0, v0, u1, v1, b0_v, b1_v, out_v, gsem, bsem):
    c = lax.axis_index("c")
    s = lax.axis_index("s")
    t = c * NSUB + s
    lane_iota = lax.iota(jnp.int32, 16)
    ubufs = [u0, u1]
    vbufs = [v0, v1]
    pltpu.sync_copy(e0_hbm.at[t], e0_v)
    pltpu.sync_copy(e1_hbm.at[t], e1_v)
    for k in range(NCK):
        pltpu.async_copy(bn_hbm.at[e0_v.at[k]], b0_v.at[k], bsem)
        pltpu.async_copy(bn_hbm.at[e1_v.at[k]], b1_v.at[k], bsem)
    pltpu.async_copy(hitem_hbm.at[e0_v.at[0]], ubufs[0], gsem)
    pltpu.async_copy(hitem_hbm.at[e1_v.at[0]], vbufs[0], gsem)
    for k in range(NCK):
        pltpu.make_async_copy(bn_hbm.at[e0_v.at[k]], b0_v.at[k], bsem).wait()
        pltpu.make_async_copy(bn_hbm.at[e1_v.at[k]], b1_v.at[k], bsem).wait()
    for k in range(NCK):
        bb = k % 2
        u, v = ubufs[bb], vbufs[bb]
        pltpu.make_async_copy(hitem_hbm.at[e0_v.at[k]], u, gsem).wait()
        pltpu.make_async_copy(hitem_hbm.at[e1_v.at[k]], v, gsem).wait()
        if k + 1 < NCK:
            nb = (k + 1) % 2
            pltpu.async_copy(hitem_hbm.at[e0_v.at[k + 1]], ubufs[nb], gsem)
            pltpu.async_copy(hitem_hbm.at[e1_v.at[k + 1]], vbufs[nb], gsem)

        def _grp16(g, _):
            res = jnp.zeros((16,), jnp.float32)
            for e in range(16):
                p = g * 16 + e
                acc = u[p, pl.ds(0, 16)] * v[p, pl.ds(0, 16)]
                for dd in range(1, D // 16):
                    sl = pl.ds(dd * 16, 16)
                    acc = acc + u[p, sl] * v[p, sl]
                sval = _hsum_lanes(acc, lane_iota)
                res = jnp.where(lane_iota == e, sval, res)
            sl16 = pl.ds(g * 16, 16)
            out_v[sl16] = res + b0_v[k, sl16] + b1_v[k, sl16]
            return 0
        lax.fori_loop(0, 8, _grp16, 0)
        pltpu.sync_copy(out_v, out_hbm.at[pl.ds(t * 640 + k * 128, 128)])


@jax.jit
def _score_edges(h_item, bn, e0r, e1r):
    mesh = plsc.VectorSubcoreMesh(core_axis_name="c", subcore_axis_name="s")
    f = pl.kernel(
        _score_body,
        out_type=jax.ShapeDtypeStruct((2 * NP,), jnp.float32),
        mesh=mesh,
        scratch_types=[
            pltpu.VMEM((NCK, 128), jnp.int32),
            pltpu.VMEM((NCK, 128), jnp.int32),
            pltpu.VMEM((128, D), jnp.float32),
            pltpu.VMEM((128, D), jnp.float32),
            pltpu.VMEM((128, D), jnp.float32),
            pltpu.VMEM((128, D), jnp.float32),
            pltpu.VMEM((NCK, 128), jnp.float32),
            pltpu.VMEM((NCK, 128), jnp.float32),
            pltpu.VMEM((128,), jnp.float32),
            pltpu.SemaphoreType.DMA,
            pltpu.SemaphoreType.DMA,
        ],
        compiler_params=pltpu.CompilerParams(use_tc_tiling_on_sc=False),
    )
    return f(h_item, bn, e0r, e1r)


def kernel(nids, edge_index1, weights1, edge_index2, weights2, pos_edges, neg_edges,
           emb_table, bias, Q1w, Q1b, W1w, W1b, Q2w, Q2b, W2w, W2b):
    nids_p = jnp.concatenate(
        [nids.astype(jnp.int32), jnp.zeros((NP - N,), jnp.int32)]
    ).reshape(NT, NCK, CS)
    h0p, bn2 = _gather_h0(emb_table, bias.reshape(N), nids_p)

    esh = (NT * NGRP, SG, CH)
    s1, d1 = edge_index1[0].reshape(esh), edge_index1[1].reshape(esh)
    s2, d2 = edge_index2[0].reshape(esh), edge_index2[1].reshape(esh)
    w1r = weights1.reshape(esh)
    w2r = weights2.reshape(esh)

    nsrc1 = _mm_relu(h0p, Q1w, Q1b)
    acc2, ws2 = _edge_agg(nsrc1, s1, d1, w1r)
    a = acc2.reshape(2, NP, D)
    w = ws2.reshape(2, NP, WSW)
    h1, nsrc2 = _combine_mm(a[0], a[1], w[0], w[1], h0p, W1w, W1b, Q2w, Q2b)
    acc2b, ws2b = _edge_agg(nsrc2, s2, d2, w2r)
    ab = acc2b.reshape(2, NP, D)
    wb = ws2b.reshape(2, NP, WSW)
    h_item = _combine(ab[0], ab[1], wb[0], wb[1], h1, W2w, W2b, h0p)

    P = pos_edges.shape[1]
    ep = jnp.concatenate([pos_edges, neg_edges], axis=1).astype(jnp.int32)
    ep = jnp.concatenate(
        [ep, jnp.zeros((2, 2 * NP - 2 * P), jnp.int32)], axis=1)
    e0r = ep[0].reshape(NT, NCK, 128)
    e1r = ep[1].reshape(NT, NCK, 128)
    scores = _score_edges(h_item, bn2.reshape(NP), e0r, e1r)
    return scores[:2 * P, None]
